# baseline (speedup 1.0000x reference)
import functools

import jax
import jax.numpy as jnp
from jax import lax
from jax.experimental import pallas as pl
from jax.experimental.pallas import tpu as pltpu
from jax.experimental.pallas import tpu_sc as plsc

NUM_ITEMS = 100000
D = 128
B = 16384
L = 16
NC = 2
NS = 16
NW = NC * NS
BPW = B // NW
GCH = 128
NG = BPW // GCH
NSL = D // L


def _permute(t, idx):
    return lax.gather(
        t, idx[:, None],
        lax.GatherDimensionNumbers(offset_dims=(), collapsed_slice_dims=(0,),
                                   start_index_map=(0,)),
        slice_sizes=(1,),
        mode=lax.GatherScatterMode.PROMISE_IN_BOUNDS)


def _sc_scores_body(ue, ie, du, ad, uf, itf, out,
                    idx_u, idx_i, du_v,
                    urows0, irows0, arows0, urows1, irows1, arows1,
                    scores_v, sem0, sem1):
    wid = lax.axis_index("s") * NC + lax.axis_index("c")
    base_row = wid * NG
    pltpu.sync_copy(du, du_v)
    pltpu.sync_copy(uf.at[pl.ds(base_row, NG)], idx_u)
    pltpu.sync_copy(itf.at[pl.ds(base_row, NG)], idx_i)
    du_sl = [du_v[pl.ds(j * L, L)] for j in range(NSL)]
    lane = lax.iota(jnp.int32, L)

    slots = [(urows0, irows0, arows0, sem0), (urows1, irows1, arows1, sem1)]

    def start(s):
        ur, ir, ar, sem = slots[s % 2]
        return (pltpu.async_copy(ue.at[idx_u.at[s]], ur, sem),
                pltpu.async_copy(ie.at[idx_i.at[s]], ir, sem),
                pltpu.async_copy(ad.at[idx_i.at[s]], ar, sem))

    pend = {0: start(0)}
    for s in range(NG):
        if s + 1 < NG:
            pend[s + 1] = start(s + 1)
        for cp in pend.pop(s):
            cp.wait()
        urows, irows, arows, _ = slots[s % 2]

        def body(g, carry, _s=s, urows=urows, irows=irows, arows=arows):
            b0 = g * L
            res = jnp.zeros((L,), jnp.float32)
            for r in range(L):
                b = b0 + r
                acc = ((urows[b, pl.ds(0, L)] + du_sl[0])
                       * (irows[b, pl.ds(0, L)] + arows[b, pl.ds(0, L)]))
                for j in range(1, NSL):
                    acc = acc + ((urows[b, pl.ds(j * L, L)] + du_sl[j])
                                 * (irows[b, pl.ds(j * L, L)]
                                    + arows[b, pl.ds(j * L, L)]))
                for k in range(4):
                    acc = acc + _permute(acc, lane ^ (1 << k))
                res = jnp.where(lane == r, acc, res)
            scores_v[pl.ds(_s * GCH + b0, L)] = res
            return carry

        lax.fori_loop(0, GCH // L, body, 0)
    pltpu.sync_copy(scores_v, out.at[pl.ds(wid * BPW, BPW)])


@functools.partial(
    pl.kernel,
    out_type=jax.ShapeDtypeStruct((B,), jnp.float32),
    mesh=plsc.VectorSubcoreMesh(core_axis_name="c", subcore_axis_name="s"),
    scratch_types=[
        pltpu.VMEM((NG, GCH), jnp.int32),
        pltpu.VMEM((NG, GCH), jnp.int32),
        pltpu.VMEM((D,), jnp.float32),
        pltpu.VMEM((GCH, D), jnp.float32),
        pltpu.VMEM((GCH, D), jnp.float32),
        pltpu.VMEM((GCH, D), jnp.float32),
        pltpu.VMEM((GCH, D), jnp.float32),
        pltpu.VMEM((GCH, D), jnp.float32),
        pltpu.VMEM((GCH, D), jnp.float32),
        pltpu.VMEM((BPW,), jnp.float32),
        pltpu.SemaphoreType.DMA,
        pltpu.SemaphoreType.DMA,
    ],
)
def _sc_scores(ue, ie, du, ad, uf, itf, out,
               idx_u, idx_i, du_v,
               urows0, irows0, arows0, urows1, irows1, arows1,
               scores_v, sem0, sem1):
    _sc_scores_body(ue, ie, du, ad, uf, itf, out,
                    idx_u, idx_i, du_v,
                    urows0, irows0, arows0, urows1, irows1, arows1,
                    scores_v, sem0, sem1)


def _tc_add_body(ie_ref, ad_ref, o_ref):
    o_ref[...] = ie_ref[...] + ad_ref[...]


_ROWS_PER_BLK = 16640


def _tc_add(item_embedding, action_delta):
    n = item_embedding.shape[0]
    return pl.pallas_call(
        _tc_add_body,
        grid=(pl.cdiv(n, _ROWS_PER_BLK),),
        in_specs=[pl.BlockSpec((_ROWS_PER_BLK, D), lambda i: (i, 0)),
                  pl.BlockSpec((_ROWS_PER_BLK, D), lambda i: (i, 0))],
        out_specs=pl.BlockSpec((_ROWS_PER_BLK, D), lambda i: (i, 0)),
        out_shape=jax.ShapeDtypeStruct((n, D), jnp.float32),
    )(item_embedding, action_delta)


def kernel(user_embedding, item_embedding, delta_user, action_delta,
           user_features, item_features):
    uf = user_features.astype(jnp.int32).reshape(B // GCH, GCH)
    itf = item_features.astype(jnp.int32).reshape(B // GCH, GCH)
    scores = _sc_scores(user_embedding, item_embedding, delta_user,
                        action_delta, uf, itf)
    item_masked = _tc_add(item_embedding, action_delta)
    return (scores, item_masked)


# speedup vs baseline: 1.0072x; 1.0072x over previous
import functools

import jax
import jax.numpy as jnp
from jax import lax
from jax.experimental import pallas as pl
from jax.experimental.pallas import tpu as pltpu
from jax.experimental.pallas import tpu_sc as plsc

NUM_ITEMS = 100000
D = 128
B = 16384
L = 16
NC = 2
NS = 16
NW = NC * NS
BPW = B // NW
GCH = 128
NG = BPW // GCH
NSL = D // L


def _permute(t, idx):
    return lax.gather(
        t, idx[:, None],
        lax.GatherDimensionNumbers(offset_dims=(), collapsed_slice_dims=(0,),
                                   start_index_map=(0,)),
        slice_sizes=(1,),
        mode=lax.GatherScatterMode.PROMISE_IN_BOUNDS)


def _sc_scores_body(ue, ie, du, ad, uf, itf, out,
                    idx_u, idx_i, du_v,
                    urows0, irows0, arows0, urows1, irows1, arows1,
                    scores_v, sem0, sem1):
    wid = lax.axis_index("s") * NC + lax.axis_index("c")
    base_row = wid * NG
    pltpu.sync_copy(du, du_v)
    pltpu.sync_copy(uf.at[pl.ds(base_row, NG)], idx_u)
    pltpu.sync_copy(itf.at[pl.ds(base_row, NG)], idx_i)
    du_sl = [du_v[pl.ds(j * L, L)] for j in range(NSL)]
    lane = lax.iota(jnp.int32, L)

    slots = [(urows0, irows0, arows0, sem0), (urows1, irows1, arows1, sem1)]

    def start(s):
        ur, ir, ar, sem = slots[s % 2]
        return (pltpu.async_copy(ue.at[idx_u.at[s]], ur, sem),
                pltpu.async_copy(ie.at[idx_i.at[s]], ir, sem),
                pltpu.async_copy(ad.at[idx_i.at[s]], ar, sem))

    pend = {0: start(0)}
    for s in range(NG):
        if s + 1 < NG:
            pend[s + 1] = start(s + 1)
        for cp in pend.pop(s):
            cp.wait()
        urows, irows, arows, _ = slots[s % 2]

        def body(g, carry, _s=s, urows=urows, irows=irows, arows=arows):
            b0 = g * L
            res = jnp.zeros((L,), jnp.float32)
            for r in range(L):
                b = b0 + r
                acc = ((urows[b, pl.ds(0, L)] + du_sl[0])
                       * (irows[b, pl.ds(0, L)] + arows[b, pl.ds(0, L)]))
                for j in range(1, NSL):
                    acc = acc + ((urows[b, pl.ds(j * L, L)] + du_sl[j])
                                 * (irows[b, pl.ds(j * L, L)]
                                    + arows[b, pl.ds(j * L, L)]))
                for k in range(4):
                    acc = acc + _permute(acc, lane ^ (1 << k))
                res = jnp.where(lane == r, acc, res)
            scores_v[pl.ds(_s * GCH + b0, L)] = res
            return carry

        lax.fori_loop(0, GCH // L, body, 0)
    pltpu.sync_copy(scores_v, out.at[pl.ds(wid * BPW, BPW)])


@functools.lru_cache(maxsize=None)
def _sc_scores_kernel():
    return pl.kernel(
        _sc_scores_body,
        out_type=jax.ShapeDtypeStruct((B,), jnp.float32),
        mesh=plsc.VectorSubcoreMesh(core_axis_name="c", subcore_axis_name="s"),
        scratch_types=[
            pltpu.VMEM((NG, GCH), jnp.int32),
            pltpu.VMEM((NG, GCH), jnp.int32),
            pltpu.VMEM((D,), jnp.float32),
            pltpu.VMEM((GCH, D), jnp.float32),
            pltpu.VMEM((GCH, D), jnp.float32),
            pltpu.VMEM((GCH, D), jnp.float32),
            pltpu.VMEM((GCH, D), jnp.float32),
            pltpu.VMEM((GCH, D), jnp.float32),
            pltpu.VMEM((GCH, D), jnp.float32),
            pltpu.VMEM((BPW,), jnp.float32),
            pltpu.SemaphoreType.DMA,
            pltpu.SemaphoreType.DMA,
        ],
    )


_CH = 2000
_DEPTH = 4


def _tc_add_body(ie_hbm, ad_hbm, o_hbm, a_v, b_v, o_v, a_s, b_s, o_s):
    n = ie_hbm.shape[0]
    nch = n // _CH
    la = _DEPTH - 1

    def in_copies(step, slot):
        row = step * _CH
        return (pltpu.make_async_copy(ie_hbm.at[pl.ds(row, _CH)],
                                      a_v.at[slot], a_s.at[slot]),
                pltpu.make_async_copy(ad_hbm.at[pl.ds(row, _CH)],
                                      b_v.at[slot], b_s.at[slot]))

    def out_copy(step, slot):
        return pltpu.make_async_copy(o_v.at[slot],
                                     o_hbm.at[pl.ds(step * _CH, _CH)],
                                     o_s.at[slot])

    for p in range(la):
        for cp in in_copies(p, p):
            cp.start()

    def body(s, carry):
        slot = lax.rem(s, _DEPTH)

        @pl.when(s + la < nch)
        def _():
            nslot = lax.rem(s + la, _DEPTH)
            for cp in in_copies(s + la, nslot):
                cp.start()

        @pl.when(s >= _DEPTH)
        def _():
            out_copy(s - _DEPTH, slot).wait()

        for cp in in_copies(s, slot):
            cp.wait()
        o_v[slot] = a_v[slot] + b_v[slot]
        out_copy(s, slot).start()
        return carry

    lax.fori_loop(0, nch, body, 0)
    for k in range(_DEPTH):
        step = nch - _DEPTH + k
        out_copy(step, step % _DEPTH).wait()


def _tc_add(item_embedding, action_delta):
    n = item_embedding.shape[0]
    return pl.pallas_call(
        _tc_add_body,
        in_specs=[pl.BlockSpec(memory_space=pl.ANY),
                  pl.BlockSpec(memory_space=pl.ANY)],
        out_specs=pl.BlockSpec(memory_space=pl.ANY),
        out_shape=jax.ShapeDtypeStruct((n, D), jnp.float32),
        scratch_shapes=[
            pltpu.VMEM((_DEPTH, _CH, D), jnp.float32),
            pltpu.VMEM((_DEPTH, _CH, D), jnp.float32),
            pltpu.VMEM((_DEPTH, _CH, D), jnp.float32),
            pltpu.SemaphoreType.DMA((_DEPTH,)),
            pltpu.SemaphoreType.DMA((_DEPTH,)),
            pltpu.SemaphoreType.DMA((_DEPTH,)),
        ],
    )(item_embedding, action_delta)


def kernel(user_embedding, item_embedding, delta_user, action_delta,
           user_features, item_features):
    uf = user_features.astype(jnp.int32).reshape(B // GCH, GCH)
    itf = item_features.astype(jnp.int32).reshape(B // GCH, GCH)
    scores = _sc_scores_kernel()(user_embedding, item_embedding, delta_user,
                                 action_delta, uf, itf)
    item_masked = _tc_add(item_embedding, action_delta)
    return (scores, item_masked)


# speedup vs baseline: 1.0138x; 1.0066x over previous
import functools

import jax
import jax.numpy as jnp
from jax import lax
from jax.experimental import pallas as pl
from jax.experimental.pallas import tpu as pltpu
from jax.experimental.pallas import tpu_sc as plsc

NUM_ITEMS = 100000
D = 128
B = 16384
L = 16
NC = 2
NS = 16
NW = NC * NS
BPW = B // NW
GCH = 128
NG = BPW // GCH
NSL = D // L


def _permute(t, idx):
    return lax.gather(
        t, idx[:, None],
        lax.GatherDimensionNumbers(offset_dims=(), collapsed_slice_dims=(0,),
                                   start_index_map=(0,)),
        slice_sizes=(1,),
        mode=lax.GatherScatterMode.PROMISE_IN_BOUNDS)


def _sc_scores_body(ue, ie, du, ad, uf, itf, out,
                    idx_u, idx_i, du_v,
                    urows0, irows0, arows0, urows1, irows1, arows1,
                    scores_v, sem0, sem1):
    wid = lax.axis_index("s") * NC + lax.axis_index("c")
    base_row = wid * NG
    pltpu.sync_copy(du, du_v)
    pltpu.sync_copy(uf.at[pl.ds(base_row, NG)], idx_u)
    pltpu.sync_copy(itf.at[pl.ds(base_row, NG)], idx_i)
    du_sl = [du_v[pl.ds(j * L, L)] for j in range(NSL)]
    lane = lax.iota(jnp.int32, L)

    slots = [(urows0, irows0, arows0, sem0), (urows1, irows1, arows1, sem1)]

    def start(s):
        ur, ir, ar, sem = slots[s % 2]
        return (pltpu.async_copy(ue.at[idx_u.at[s]], ur, sem),
                pltpu.async_copy(ie.at[idx_i.at[s]], ir, sem),
                pltpu.async_copy(ad.at[idx_i.at[s]], ar, sem))

    pend = {0: start(0)}
    for s in range(NG):
        if s + 1 < NG:
            pend[s + 1] = start(s + 1)
        for cp in pend.pop(s):
            cp.wait()
        urows, irows, arows, _ = slots[s % 2]

        def body(g, carry, _s=s, urows=urows, irows=irows, arows=arows):
            b0 = g * L
            res = jnp.zeros((L,), jnp.float32)
            for r in range(L):
                b = b0 + r
                acc = ((urows[b, pl.ds(0, L)] + du_sl[0])
                       * (irows[b, pl.ds(0, L)] + arows[b, pl.ds(0, L)]))
                for j in range(1, NSL):
                    acc = acc + ((urows[b, pl.ds(j * L, L)] + du_sl[j])
                                 * (irows[b, pl.ds(j * L, L)]
                                    + arows[b, pl.ds(j * L, L)]))
                for k in range(4):
                    acc = acc + _permute(acc, lane ^ (1 << k))
                res = jnp.where(lane == r, acc, res)
            scores_v[pl.ds(_s * GCH + b0, L)] = res
            return carry

        lax.fori_loop(0, GCH // L, body, 0)
    pltpu.sync_copy(scores_v, out.at[pl.ds(wid * BPW, BPW)])


@functools.lru_cache(maxsize=None)
def _sc_scores_kernel():
    return pl.kernel(
        _sc_scores_body,
        out_type=jax.ShapeDtypeStruct((B,), jnp.float32),
        mesh=plsc.VectorSubcoreMesh(core_axis_name="c", subcore_axis_name="s"),
        scratch_types=[
            pltpu.VMEM((NG, GCH), jnp.int32),
            pltpu.VMEM((NG, GCH), jnp.int32),
            pltpu.VMEM((D,), jnp.float32),
            pltpu.VMEM((GCH, D), jnp.float32),
            pltpu.VMEM((GCH, D), jnp.float32),
            pltpu.VMEM((GCH, D), jnp.float32),
            pltpu.VMEM((GCH, D), jnp.float32),
            pltpu.VMEM((GCH, D), jnp.float32),
            pltpu.VMEM((GCH, D), jnp.float32),
            pltpu.VMEM((BPW,), jnp.float32),
            pltpu.SemaphoreType.DMA,
            pltpu.SemaphoreType.DMA,
        ],
    )


def _tc_add_body(ie_ref, ad_ref, o_ref):
    o_ref[...] = ie_ref[...] + ad_ref[...]


_ROWS_PER_BLK = 12800


def _tc_add(item_embedding, action_delta):
    n = item_embedding.shape[0]
    return pl.pallas_call(
        _tc_add_body,
        grid=(pl.cdiv(n, _ROWS_PER_BLK),),
        in_specs=[pl.BlockSpec((_ROWS_PER_BLK, D), lambda i: (i, 0)),
                  pl.BlockSpec((_ROWS_PER_BLK, D), lambda i: (i, 0))],
        out_specs=pl.BlockSpec((_ROWS_PER_BLK, D), lambda i: (i, 0)),
        out_shape=jax.ShapeDtypeStruct((n, D), jnp.float32),
    )(item_embedding, action_delta)


def kernel(user_embedding, item_embedding, delta_user, action_delta,
           user_features, item_features):
    uf = user_features.astype(jnp.int32).reshape(B // GCH, GCH)
    itf = item_features.astype(jnp.int32).reshape(B // GCH, GCH)
    scores = _sc_scores_kernel()(user_embedding, item_embedding, delta_user,
                                 action_delta, uf, itf)
    item_masked = _tc_add(item_embedding, action_delta)
    return (scores, item_masked)
